# baseline (speedup 1.0000x reference)
import functools

import jax
import jax.numpy as jnp
from jax import lax
from jax.experimental import pallas as pl
from jax.experimental.pallas import tpu as pltpu

GRID_RES = 128
NCELL = GRID_RES ** 3
_F32 = jnp.float32


def _prep_body(xq_ref, xr_ref, xc_ref, w_ref, qb_ref, qf_ref, sb_ref, rec_ref):
    r = GRID_RES - 1

    xq = jnp.clip(xq_ref[...], 0.0, 1.0)
    qc = xq * float(r)
    qb = jnp.minimum(jnp.floor(qc), float(r - 1))
    qf_ref[...] = qc - qb
    qbi = qb.astype(jnp.int32)
    qb_ref[...] = (qbi[0] * (GRID_RES * GRID_RES) + qbi[1] * GRID_RES + qbi[2])

    xr = jnp.clip(xr_ref[...], 0.0, 1.0)
    u = xc_ref[...] - xr
    f = []
    for i in range(3):
        acc = w_ref[3, i]
        for j in range(3):
            acc = acc + u[j] * w_ref[j, i]
        f.append(jax.nn.gelu(acc))
    sc = jnp.clip(xr * float(r), 0.0, float(r) - 0.001)
    sbf = jnp.floor(sc)
    sd = sc - sbf
    sbi = sbf.astype(jnp.int32)
    sb_ref[...] = (sbi[0] * (GRID_RES * GRID_RES) + sbi[1] * GRID_RES + sbi[2])
    rec_ref[...] = jnp.stack(
        [f[0], f[1], f[2], u[0], u[1], u[2], sd[0], sd[1], sd[2]], axis=0)


def _prep(xq3, xr3, xc3, w8):
    nb = xq3.shape[1]
    bm = min(512, nb)
    grid = (nb // bm,)
    csp = pl.BlockSpec((3, bm, 128), lambda i: (0, i, 0))
    isp = pl.BlockSpec((bm, 128), lambda i: (i, 0))
    return pl.pallas_call(
        _prep_body,
        grid=grid,
        in_specs=[csp, csp, csp, pl.BlockSpec((8, 8), lambda i: (0, 0))],
        out_specs=[isp, csp, isp, pl.BlockSpec((9, bm, 128), lambda i: (0, i, 0))],
        out_shape=[
            jax.ShapeDtypeStruct((nb, 128), jnp.int32),
            jax.ShapeDtypeStruct((3, nb, 128), _F32),
            jax.ShapeDtypeStruct((nb, 128), jnp.int32),
            jax.ShapeDtypeStruct((9, nb, 128), _F32),
        ],
    )(xq3, xr3, xc3, w8)


def _decode_body(raw_ref, qf_ref, w1_ref, w2_ref, wo_ref, out_ref):
    raw = raw_ref[...]
    f = raw[0:3]
    dens = raw[3:4]
    u = raw[4:7]
    denom = jnp.maximum(dens, 1e-05)
    mask = (dens > 1e-05).astype(_F32)
    scale = mask / denom
    fn = f * scale
    un = u * scale
    qf = qf_ref[...]
    pe = []
    for i in range(3):
        freq = (2.0 ** i) * jnp.pi
        pe.append(jnp.sin(freq * qf))
        pe.append(jnp.cos(freq * qf))
    b = raw.shape[1]
    x = jnp.concatenate(
        [fn, un] + pe + [jnp.ones((1, b), _F32), jnp.zeros((7, b), _F32)],
        axis=0)
    h = jax.nn.gelu(jnp.dot(w1_ref[...], x, preferred_element_type=_F32))
    h = jnp.concatenate([h, jnp.ones((1, b), _F32), jnp.zeros((7, b), _F32)],
                        axis=0)
    h = jax.nn.gelu(jnp.dot(w2_ref[...], h, preferred_element_type=_F32))
    h = jnp.concatenate([h, jnp.ones((1, b), _F32), jnp.zeros((7, b), _F32)],
                        axis=0)
    o = jnp.dot(wo_ref[...], h, preferred_element_type=_F32)
    drift = o[0:3] + un
    out_ref[...] = jnp.clip(
        jnp.concatenate([drift, jnp.zeros((5, b), _F32)], axis=0), 0.001, 0.999)


def _decode(raw_t, qf3, w1a, w2a, woa):
    np_ = raw_t.shape[1]
    b = 2048
    grid = (np_ // b,)
    return pl.pallas_call(
        _decode_body,
        grid=grid,
        in_specs=[
            pl.BlockSpec((8, b), lambda i: (0, i)),
            pl.BlockSpec((3, b), lambda i: (0, i)),
            pl.BlockSpec((64, 32), lambda i: (0, 0)),
            pl.BlockSpec((64, 72), lambda i: (0, 0)),
            pl.BlockSpec((8, 72), lambda i: (0, 0)),
        ],
        out_specs=pl.BlockSpec((8, b), lambda i: (0, i)),
        out_shape=jax.ShapeDtypeStruct((8, np_), _F32),
    )(raw_t, qf3, w1a, w2a, woa)


def kernel(x_query_ref, x_source_ref, x_source_curr, W_trans, b_trans,
           W_dec1, b_dec1, W_dec2, b_dec2, W_out, b_out):
    nq = x_query_ref.shape[0]
    ns = x_source_ref.shape[0]
    npad = 1 << 14
    while npad < max(nq, ns):
        npad *= 2

    def cm(x, n):
        pad = jnp.full((npad - n, 3), 0.5, _F32)
        return jnp.concatenate([x, pad], 0).T.reshape(3, npad // 128, 128)

    xq3 = cm(x_query_ref, nq)
    xr3 = cm(x_source_ref, ns)
    xc3 = cm(x_source_curr, ns)

    w8 = jnp.zeros((8, 8), _F32).at[:3, :3].set(W_trans).at[3, :3].set(b_trans)
    qb, qf3, sb, rec = _prep(xq3, xr3, xc3, w8)

    qb_f = qb.reshape(npad)
    sb_f = sb.reshape(npad)
    rec_f = rec.reshape(9, npad)

    valid = (jnp.arange(npad, dtype=jnp.int32) < ns).astype(_F32)
    content = jnp.concatenate(
        [rec_f[0:3], valid[None], rec_f[3:6]], axis=0)
    d = rec_f[6:9]
    grid_acc = jnp.zeros((NCELL, 7), _F32)
    for off in range(8):
        ox, oy, oz = (off >> 2) & 1, (off >> 1) & 1, off & 1
        w = ((d[0] if ox else (1.0 - d[0]))
             * (d[1] if oy else (1.0 - d[1]))
             * (d[2] if oz else (1.0 - d[2]))) * valid
        idx = sb_f + (ox * GRID_RES * GRID_RES + oy * GRID_RES + oz)
        grid_acc = grid_acc.at[idx].add((content * w).T)

    qd = qf3.reshape(3, npad)
    raw = jnp.zeros((npad, 7), _F32)
    for off in range(8):
        ox, oy, oz = (off >> 2) & 1, (off >> 1) & 1, off & 1
        w = ((qd[0] if ox else (1.0 - qd[0]))
             * (qd[1] if oy else (1.0 - qd[1]))
             * (qd[2] if oz else (1.0 - qd[2])))
        idx = qb_f + (ox * GRID_RES * GRID_RES + oy * GRID_RES + oz)
        raw = raw + grid_acc[idx] * w[:, None]

    raw_t = jnp.concatenate([raw.T, jnp.zeros((1, npad), _F32)], axis=0)

    w1a = jnp.zeros((64, 32), _F32).at[:, :24].set(W_dec1.T).at[:, 24].set(b_dec1)
    w2a = jnp.zeros((64, 72), _F32).at[:, :64].set(W_dec2.T).at[:, 64].set(b_dec2)
    woa = jnp.zeros((8, 72), _F32).at[:3, :64].set(W_out.T).at[:3, 64].set(b_out)
    qcoord = jnp.clip(x_query_ref, 0.0, 1.0).T
    qcoord = jnp.concatenate(
        [qcoord, jnp.full((3, npad - nq), 0.5, _F32)], axis=1)
    out8 = _decode(raw_t, qcoord, w1a, w2a, woa)
    return out8[:3, :nq].T


# speedup vs baseline: 1.0538x; 1.0538x over previous
import functools

import jax
import jax.numpy as jnp
from jax import lax
from jax.experimental import pallas as pl
from jax.experimental.pallas import tpu as pltpu
from jax.experimental.pallas import tpu_sc as plsc

GRID_RES = 128
NCELL = GRID_RES ** 3
_F32 = jnp.float32


def _corner_iw(base_flat, d, c):
    ox, oy, oz = (c >> 2) & 1, (c >> 1) & 1, c & 1
    idx = base_flat + (ox * GRID_RES * GRID_RES + oy * GRID_RES + oz)
    w = ((d[0] if ox else (1.0 - d[0]))
         * (d[1] if oy else (1.0 - d[1]))
         * (d[2] if oz else (1.0 - d[2])))
    return idx, w


def _prep_body(xq_ref, xr_ref, xc_ref, w_ref, qi_ref, qw_ref, sb_ref, rec_ref):
    r = GRID_RES - 1

    xq = jnp.clip(xq_ref[...], 0.0, 1.0)
    qc = xq * float(r)
    qb = jnp.minimum(jnp.floor(qc), float(r - 1))
    qd = qc - qb
    qbi = qb.astype(jnp.int32)
    qflat = (qbi[0] * (GRID_RES * GRID_RES) + qbi[1] * GRID_RES + qbi[2])
    qis, qws = [], []
    for c in range(8):
        i, w = _corner_iw(qflat, qd, c)
        qis.append(i)
        qws.append(w)
    qi_ref[...] = jnp.stack(qis, axis=0)
    qw_ref[...] = jnp.stack(qws, axis=0)

    xr = jnp.clip(xr_ref[...], 0.0, 1.0)
    u = xc_ref[...] - xr
    f = []
    for i in range(3):
        acc = w_ref[3, i]
        for j in range(3):
            acc = acc + u[j] * w_ref[j, i]
        f.append(jax.nn.gelu(acc))
    sc = jnp.clip(xr * float(r), 0.0, float(r) - 0.001)
    sbf = jnp.floor(sc)
    sd = sc - sbf
    sbi = sbf.astype(jnp.int32)
    sb_ref[...] = (sbi[0] * (GRID_RES * GRID_RES) + sbi[1] * GRID_RES + sbi[2])
    rec_ref[...] = jnp.stack(
        [f[0], f[1], f[2], u[0], u[1], u[2], sd[0], sd[1], sd[2]], axis=0)


def _prep(xq3, xr3, xc3, w8):
    nb = xq3.shape[1]
    bm = min(512, nb)
    grid = (nb // bm,)
    csp = pl.BlockSpec((3, bm, 128), lambda i: (0, i, 0))
    isp = pl.BlockSpec((bm, 128), lambda i: (i, 0))
    c8sp = pl.BlockSpec((8, bm, 128), lambda i: (0, i, 0))
    return pl.pallas_call(
        _prep_body,
        grid=grid,
        in_specs=[csp, csp, csp, pl.BlockSpec((8, 8), lambda i: (0, 0))],
        out_specs=[c8sp, c8sp, isp, pl.BlockSpec((9, bm, 128), lambda i: (0, i, 0))],
        out_shape=[
            jax.ShapeDtypeStruct((8, nb, 128), jnp.int32),
            jax.ShapeDtypeStruct((8, nb, 128), _F32),
            jax.ShapeDtypeStruct((nb, 128), jnp.int32),
            jax.ShapeDtypeStruct((9, nb, 128), _F32),
        ],
    )(xq3, xr3, xc3, w8)


def _decode_body(raw_ref, qf_ref, w1_ref, w2_ref, wo_ref, out_ref):
    raw = raw_ref[...]
    f = raw[0:3]
    dens = raw[3:4]
    u = raw[4:7]
    denom = jnp.maximum(dens, 1e-05)
    mask = (dens > 1e-05).astype(_F32)
    scale = mask / denom
    fn = f * scale
    un = u * scale
    qf = qf_ref[...]
    pe = []
    for i in range(3):
        freq = (2.0 ** i) * jnp.pi
        pe.append(jnp.sin(freq * qf))
        pe.append(jnp.cos(freq * qf))
    b = raw.shape[1]
    x = jnp.concatenate(
        [fn, un] + pe + [jnp.ones((1, b), _F32), jnp.zeros((7, b), _F32)],
        axis=0)
    h = jax.nn.gelu(jnp.dot(w1_ref[...], x, preferred_element_type=_F32))
    h = jnp.concatenate([h, jnp.ones((1, b), _F32), jnp.zeros((7, b), _F32)],
                        axis=0)
    h = jax.nn.gelu(jnp.dot(w2_ref[...], h, preferred_element_type=_F32))
    h = jnp.concatenate([h, jnp.ones((1, b), _F32), jnp.zeros((7, b), _F32)],
                        axis=0)
    o = jnp.dot(wo_ref[...], h, preferred_element_type=_F32)
    drift = o[0:3] + un
    out_ref[...] = jnp.clip(
        jnp.concatenate([drift, jnp.zeros((5, b), _F32)], axis=0), 0.001, 0.999)


def _decode(raw_t, qf3, w1a, w2a, woa):
    np_ = raw_t.shape[1]
    b = 2048
    grid = (np_ // b,)
    return pl.pallas_call(
        _decode_body,
        grid=grid,
        in_specs=[
            pl.BlockSpec((8, b), lambda i: (0, i)),
            pl.BlockSpec((3, b), lambda i: (0, i)),
            pl.BlockSpec((64, 32), lambda i: (0, 0)),
            pl.BlockSpec((64, 72), lambda i: (0, 0)),
            pl.BlockSpec((8, 72), lambda i: (0, 0)),
        ],
        out_specs=pl.BlockSpec((8, b), lambda i: (0, i)),
        out_shape=jax.ShapeDtypeStruct((8, np_), _F32),
    )(raw_t, qf3, w1a, w2a, woa)


def _vperm(v, idx):
    return lax.gather(
        v, idx[:, None],
        lax.GatherDimensionNumbers(offset_dims=(), collapsed_slice_dims=(0,),
                                   start_index_map=(0,)),
        (1,), mode=lax.GatherScatterMode.PROMISE_IN_BOUNDS)


def _sc_sample(grid8, qidx3, qw3):
    nb = qidx3.shape[1]
    npnt = nb * 128
    nc, ns = 2, 16
    nw = nc * ns
    per_w = npnt // nw
    k = min(512, per_w)
    kr = k // 128
    nchunk = per_w // k
    mesh = plsc.VectorSubcoreMesh(core_axis_name="c", subcore_axis_name="s")

    @functools.partial(
        pl.kernel, mesh=mesh,
        out_type=jax.ShapeDtypeStruct((npnt, 16), _F32),
        compiler_params=pltpu.CompilerParams(use_tc_tiling_on_sc=False),
        scratch_types=[
            pltpu.VMEM((8, kr, 128), jnp.int32),
            pltpu.VMEM((8, kr, 128), _F32),
            pltpu.VMEM((8, k, 16), _F32),
            pltpu.VMEM((k, 16), _F32),
            pltpu.SemaphoreType.DMA,
        ])
    def body(grid_hbm, qi_hbm, qw_hbm, out_hbm, idxb, wbuf, rows, accb, sem):
        wid = lax.axis_index("s") * nc + lax.axis_index("c")
        base = wid * per_w
        bcast = [jnp.full((16,), l, jnp.int32) for l in range(16)]

        def chunk(ci, carry):
            col = base + ci * k
            rb = pl.multiple_of(col // 128, kr)
            for c in range(8):
                pltpu.sync_copy(qi_hbm.at[c, pl.ds(rb, kr), :], idxb.at[c])
                pltpu.sync_copy(qw_hbm.at[c, pl.ds(rb, kr), :], wbuf.at[c])
            descs = [pltpu.async_copy(grid_hbm.at[idxb.at[c, s]],
                                      rows.at[c, pl.ds(s * 128, 128), :], sem)
                     for c in range(8) for s in range(kr)]
            for d_ in descs:
                d_.wait()

            def group(g, carry2):
                r = g // 8
                coloff = (g % 8) * 16
                wfull = [wbuf[c, r, pl.ds(coloff, 16)] for c in range(8)]
                for lq in range(16):
                    q = g * 16 + lq
                    acc = jnp.zeros((16,), _F32)
                    for c in range(8):
                        acc = acc + _vperm(wfull[c], bcast[lq]) * rows[c, q, :]
                    accb[q, :] = acc
                return carry2

            lax.fori_loop(0, k // 16, group, 0)
            pltpu.sync_copy(accb, out_hbm.at[pl.ds(col, k), :])
            return carry

        lax.fori_loop(0, nchunk, chunk, 0)

    return body(grid8, qidx3, qw3)


def kernel(x_query_ref, x_source_ref, x_source_curr, W_trans, b_trans,
           W_dec1, b_dec1, W_dec2, b_dec2, W_out, b_out):
    nq = x_query_ref.shape[0]
    ns = x_source_ref.shape[0]
    npad = 1 << 14
    while npad < max(nq, ns):
        npad *= 2

    def cm(x, n):
        pad = jnp.full((npad - n, 3), 0.5, _F32)
        return jnp.concatenate([x, pad], 0).T.reshape(3, npad // 128, 128)

    xq3 = cm(x_query_ref, nq)
    xr3 = cm(x_source_ref, ns)
    xc3 = cm(x_source_curr, ns)

    w8 = jnp.zeros((8, 8), _F32).at[:3, :3].set(W_trans).at[3, :3].set(b_trans)
    qidx8, qw8, sb, rec = _prep(xq3, xr3, xc3, w8)

    sb_f = sb.reshape(npad)
    rec_f = rec.reshape(9, npad)

    valid = (jnp.arange(npad, dtype=jnp.int32) < ns).astype(_F32)
    content = jnp.concatenate(
        [rec_f[0:3], valid[None], rec_f[3:6]], axis=0)
    d = rec_f[6:9]
    grid_acc = jnp.zeros((NCELL, 7), _F32)
    for off in range(8):
        ox, oy, oz = (off >> 2) & 1, (off >> 1) & 1, off & 1
        w = ((d[0] if ox else (1.0 - d[0]))
             * (d[1] if oy else (1.0 - d[1]))
             * (d[2] if oz else (1.0 - d[2]))) * valid
        idx = sb_f + (ox * GRID_RES * GRID_RES + oy * GRID_RES + oz)
        grid_acc = grid_acc.at[idx].add((content * w).T)

    grid16 = jnp.pad(grid_acc, ((0, 0), (0, 9)))
    raw_pm = _sc_sample(grid16, qidx8, qw8)
    raw_t = raw_pm.T[:8]

    w1a = jnp.zeros((64, 32), _F32).at[:, :24].set(W_dec1.T).at[:, 24].set(b_dec1)
    w2a = jnp.zeros((64, 72), _F32).at[:, :64].set(W_dec2.T).at[:, 64].set(b_dec2)
    woa = jnp.zeros((8, 72), _F32).at[:3, :64].set(W_out.T).at[:3, 64].set(b_out)
    qcoord = jnp.clip(x_query_ref, 0.0, 1.0).T
    qcoord = jnp.concatenate(
        [qcoord, jnp.full((3, npad - nq), 0.5, _F32)], axis=1)
    out8 = _decode(raw_t, qcoord, w1a, w2a, woa)
    return out8[:3, :nq].T


# speedup vs baseline: 3.6268x; 3.4417x over previous
import functools

import jax
import jax.numpy as jnp
from jax import lax
from jax.experimental import pallas as pl
from jax.experimental.pallas import tpu as pltpu
from jax.experimental.pallas import tpu_sc as plsc

GRID_RES = 128
NCELL = GRID_RES ** 3
_F32 = jnp.float32


def _corner_iw(base_flat, d, c):
    ox, oy, oz = (c >> 2) & 1, (c >> 1) & 1, c & 1
    idx = base_flat + (ox * GRID_RES * GRID_RES + oy * GRID_RES + oz)
    w = ((d[0] if ox else (1.0 - d[0]))
         * (d[1] if oy else (1.0 - d[1]))
         * (d[2] if oz else (1.0 - d[2])))
    return idx, w


def _prep_body(xq_ref, xr_ref, xc_ref, w_ref, qi_ref, qw_ref, sb_ref, rec_ref):
    r = GRID_RES - 1

    xq = jnp.clip(xq_ref[...], 0.0, 1.0)
    qc = xq * float(r)
    qb = jnp.minimum(jnp.floor(qc), float(r - 1))
    qd = qc - qb
    qbi = qb.astype(jnp.int32)
    qflat = (qbi[0] * (GRID_RES * GRID_RES) + qbi[1] * GRID_RES + qbi[2])
    qis, qws = [], []
    for c in range(8):
        i, w = _corner_iw(qflat, qd, c)
        qis.append(i)
        qws.append(w)
    qi_ref[...] = jnp.stack(qis, axis=0)
    qw_ref[...] = jnp.stack(qws, axis=0)

    xr = jnp.clip(xr_ref[...], 0.0, 1.0)
    u = xc_ref[...] - xr
    f = []
    for i in range(3):
        acc = w_ref[3, i]
        for j in range(3):
            acc = acc + u[j] * w_ref[j, i]
        f.append(jax.nn.gelu(acc))
    sc = jnp.clip(xr * float(r), 0.0, float(r) - 0.001)
    sbf = jnp.floor(sc)
    sd = sc - sbf
    sbi = sbf.astype(jnp.int32)
    sb_ref[...] = (sbi[0] * (GRID_RES * GRID_RES) + sbi[1] * GRID_RES + sbi[2])
    rec_ref[...] = jnp.stack(
        [f[0], f[1], f[2], u[0], u[1], u[2], sd[0], sd[1], sd[2]], axis=0)


def _prep(xq3, xr3, xc3, w8):
    nb = xq3.shape[1]
    bm = min(512, nb)
    grid = (nb // bm,)
    csp = pl.BlockSpec((3, bm, 128), lambda i: (0, i, 0))
    isp = pl.BlockSpec((bm, 128), lambda i: (i, 0))
    c8sp = pl.BlockSpec((8, bm, 128), lambda i: (0, i, 0))
    return pl.pallas_call(
        _prep_body,
        grid=grid,
        in_specs=[csp, csp, csp, pl.BlockSpec((8, 8), lambda i: (0, 0))],
        out_specs=[c8sp, c8sp, isp, pl.BlockSpec((9, bm, 128), lambda i: (0, i, 0))],
        out_shape=[
            jax.ShapeDtypeStruct((8, nb, 128), jnp.int32),
            jax.ShapeDtypeStruct((8, nb, 128), _F32),
            jax.ShapeDtypeStruct((nb, 128), jnp.int32),
            jax.ShapeDtypeStruct((9, nb, 128), _F32),
        ],
    )(xq3, xr3, xc3, w8)


def _decode_body(raw_ref, qf_ref, w1_ref, w2_ref, wo_ref, out_ref):
    raw = raw_ref[...]
    f = raw[0:3]
    dens = raw[3:4]
    u = raw[4:7]
    denom = jnp.maximum(dens, 1e-05)
    mask = (dens > 1e-05).astype(_F32)
    scale = mask / denom
    fn = f * scale
    un = u * scale
    qf = qf_ref[...]
    pe = []
    for i in range(3):
        freq = (2.0 ** i) * jnp.pi
        pe.append(jnp.sin(freq * qf))
        pe.append(jnp.cos(freq * qf))
    b = raw.shape[1]
    x = jnp.concatenate(
        [fn, un] + pe + [jnp.ones((1, b), _F32), jnp.zeros((7, b), _F32)],
        axis=0)
    h = jax.nn.gelu(jnp.dot(w1_ref[...], x, preferred_element_type=_F32))
    h = jnp.concatenate([h, jnp.ones((1, b), _F32), jnp.zeros((7, b), _F32)],
                        axis=0)
    h = jax.nn.gelu(jnp.dot(w2_ref[...], h, preferred_element_type=_F32))
    h = jnp.concatenate([h, jnp.ones((1, b), _F32), jnp.zeros((7, b), _F32)],
                        axis=0)
    o = jnp.dot(wo_ref[...], h, preferred_element_type=_F32)
    drift = o[0:3] + un
    out_ref[...] = jnp.clip(
        jnp.concatenate([drift, jnp.zeros((5, b), _F32)], axis=0), 0.001, 0.999)


def _decode(raw_t, qf3, w1a, w2a, woa):
    np_ = raw_t.shape[1]
    b = 2048
    grid = (np_ // b,)
    return pl.pallas_call(
        _decode_body,
        grid=grid,
        in_specs=[
            pl.BlockSpec((8, b), lambda i: (0, i)),
            pl.BlockSpec((3, b), lambda i: (0, i)),
            pl.BlockSpec((64, 32), lambda i: (0, 0)),
            pl.BlockSpec((64, 72), lambda i: (0, 0)),
            pl.BlockSpec((8, 72), lambda i: (0, 0)),
        ],
        out_specs=pl.BlockSpec((8, b), lambda i: (0, i)),
        out_shape=jax.ShapeDtypeStruct((8, np_), _F32),
    )(raw_t, qf3, w1a, w2a, woa)


def _vperm(v, idx):
    return lax.gather(
        v, idx[:, None],
        lax.GatherDimensionNumbers(offset_dims=(), collapsed_slice_dims=(0,),
                                   start_index_map=(0,)),
        (1,), mode=lax.GatherScatterMode.PROMISE_IN_BOUNDS)


_OFF = [0, 1, 128, 129, 16384, 16385, 16512, 16513]


def _sc_scatter(sb1d, rec2, n_src):
    npnt = sb1d.shape[0]
    per_t = npnt // 16
    k = min(2048, per_t)
    nchunk = per_t // k
    p_sz = 90880
    npart = 24
    npass = 12
    tail = NCELL - (npart - 1) * p_sz
    e = 1024
    zrows = (p_sz + 128) // 16
    mesh = plsc.VectorSubcoreMesh(core_axis_name="c", subcore_axis_name="s")

    @functools.partial(
        pl.kernel, mesh=mesh,
        out_type=jax.ShapeDtypeStruct((NCELL, 16), _F32),
        compiler_params=pltpu.CompilerParams(use_tc_tiling_on_sc=False,
                                             needs_layout_passes=False),
        scratch_types=[
            pltpu.VMEM((k,), jnp.int32),
            pltpu.VMEM((9, k), _F32),
            pltpu.VMEM((k + 16,), jnp.int32),
            pltpu.VMEM((e, 16), _F32),
            pltpu.VMEM((8, 128), jnp.int32),
            pltpu.VMEM_SHARED((p_sz + 128, 16), _F32),
            pltpu.SemaphoreType.DMA,
        ])
    def body(sb_hbm, rec_hbm, grid_hbm, idxs, chs, liveb, vals, eidx,
             accum, sem):
        ci = lax.axis_index("c")
        sid = lax.axis_index("s")
        lane = lax.iota(jnp.int32, 16)
        dump = [jnp.full((16,), p_sz + c * 16, jnp.int32) + lane
                for c in range(8)]

        def one_pass(pa, carry0):
            p = pa * 2 + ci
            lo = p * p_sz
            hi = jnp.minimum(lo + p_sz, NCELL)
            psize = hi - lo

            @pl.when(p < npart)
            def _run():
                def vrow(i, carry):
                    vals[i, :] = jnp.zeros((16,), _F32)
                    return carry
                lax.fori_loop(0, e, vrow, 0)
                for i in range(5):
                    pltpu.sync_copy(
                        vals.at[pl.ds(0, 1024), :],
                        accum.at[pl.ds(sid * zrows + i * 1024, 1024), :])
                pltpu.sync_copy(
                    vals.at[pl.ds(0, 568), :],
                    accum.at[pl.ds(sid * zrows + 5120, 568), :])
                plsc.subcore_barrier()

                def chunk(ci2, carry):
                    cstart = sid * per_t + ci2 * k
                    pltpu.sync_copy(sb_hbm.at[pl.ds(cstart, k)], idxs)
                    for c in range(9):
                        pltpu.sync_copy(rec_hbm.at[c].at[pl.ds(cstart, k)],
                                        chs.at[c])

                    def scan(g, nlive):
                        bv = idxs[pl.ds(g * 16, 16)]
                        ordv = cstart + g * 16 + lane
                        live = ((bv >= lo - 16513) & (bv < hi)
                                & (ordv < n_src))
                        cs = plsc.cumsum(jnp.where(live, 1, 0))
                        plsc.store_scatter(liveb, [nlive + cs - 1],
                                           g * 16 + lane, mask=live)
                        return nlive + jnp.sum(jnp.where(live, 1, 0))

                    nlive = lax.fori_loop(0, k // 16, scan, 0)
                    ntrip = (nlive + 15) // 16

                    def livegrp(t, carry2):
                        ordv = liveb[pl.ds(t * 16, 16)]
                        lmask = (t * 16 + lane) < nlive
                        ordv = jnp.where(lmask, ordv, 0)
                        bvl = plsc.load_gather(idxs, [ordv]) - lo
                        ch = [plsc.load_gather(chs.at[c], [ordv])
                              for c in range(9)]
                        dx, dy, dz = ch[6], ch[7], ch[8]
                        wx0, wy0, wz0 = 1.0 - dx, 1.0 - dy, 1.0 - dz
                        wxy = [wx0 * wy0, wx0 * dy, dx * wy0, dx * dy]
                        blk = t & 7
                        blkv = jnp.full((16,), blk, jnp.int32)
                        for c in range(8):
                            w = wxy[c >> 1] * (dz if (c & 1) else wz0)
                            cell = bvl + _OFF[c]
                            inm = (cell >= 0) & (cell < psize) & lmask
                            cellw = jnp.where(inm, cell, dump[c])
                            col = jnp.full((16,), c * 16, jnp.int32) + lane
                            plsc.store_scatter(eidx, [blkv, col], cellw)
                            pos = blkv * 128 + col
                            vj = [ch[0] * w, ch[1] * w, ch[2] * w, w,
                                  ch[3] * w, ch[4] * w, ch[5] * w]
                            for j in range(7):
                                plsc.store_scatter(
                                    vals, [pos, jnp.full((16,), j, jnp.int32)],
                                    vj[j])

                        @pl.when(blk == 7)
                        def _flush():
                            for f in range(8):
                                pltpu.sync_copy(
                                    vals.at[pl.ds(f * 128, 128), :],
                                    accum.at[eidx.at[f]], add=True)
                        return carry2

                    lax.fori_loop(0, ntrip, livegrp, 0)

                    def fflush(f, carry3):
                        pltpu.sync_copy(vals.at[pl.ds(f * 128, 128), :],
                                        accum.at[eidx.at[f]], add=True)
                        return carry3

                    lax.fori_loop(0, ntrip & 7, fflush, 0)
                    return carry

                lax.fori_loop(0, nchunk, chunk, 0)
                plsc.subcore_barrier()

                @pl.when(p == npart - 1)
                def _dlast():
                    tr = tail // 16
                    pltpu.sync_copy(
                        accum.at[pl.ds(sid * tr, tr), :],
                        grid_hbm.at[pl.ds(lo + sid * tr, tr), :])

                @pl.when(p < npart - 1)
                def _dfull():
                    for i in range(5):
                        pltpu.sync_copy(
                            accum.at[pl.ds(sid * 5680 + i * 1024, 1024), :],
                            grid_hbm.at[pl.ds(lo + sid * 5680 + i * 1024,
                                              1024), :])
                    pltpu.sync_copy(
                        accum.at[pl.ds(sid * 5680 + 5120, 560), :],
                        grid_hbm.at[pl.ds(lo + sid * 5680 + 5120, 560), :])
            return carry0

        lax.fori_loop(0, npass, one_pass, 0)

    return body(sb1d, rec2)
def _sc_sample(grid8, qidx3, qw3):
    nb = qidx3.shape[1]
    npnt = nb * 128
    nc, ns = 2, 16
    nw = nc * ns
    per_w = npnt // nw
    k = min(512, per_w)
    kr = k // 128
    nchunk = per_w // k
    mesh = plsc.VectorSubcoreMesh(core_axis_name="c", subcore_axis_name="s")

    @functools.partial(
        pl.kernel, mesh=mesh,
        out_type=jax.ShapeDtypeStruct((npnt, 16), _F32),
        compiler_params=pltpu.CompilerParams(use_tc_tiling_on_sc=False),
        scratch_types=[
            pltpu.VMEM((8, kr, 128), jnp.int32),
            pltpu.VMEM((8, kr, 128), _F32),
            pltpu.VMEM((8, k, 16), _F32),
            pltpu.VMEM((k, 16), _F32),
            pltpu.SemaphoreType.DMA,
        ])
    def body(grid_hbm, qi_hbm, qw_hbm, out_hbm, idxb, wbuf, rows, accb, sem):
        wid = lax.axis_index("s") * nc + lax.axis_index("c")
        base = wid * per_w
        bcast = [jnp.full((16,), l, jnp.int32) for l in range(16)]

        def chunk(ci, carry):
            col = base + ci * k
            rb = pl.multiple_of(col // 128, kr)
            for c in range(8):
                pltpu.sync_copy(qi_hbm.at[c, pl.ds(rb, kr), :], idxb.at[c])
                pltpu.sync_copy(qw_hbm.at[c, pl.ds(rb, kr), :], wbuf.at[c])
            descs = [pltpu.async_copy(grid_hbm.at[idxb.at[c, s]],
                                      rows.at[c, pl.ds(s * 128, 128), :], sem)
                     for c in range(8) for s in range(kr)]
            for d_ in descs:
                d_.wait()

            def group(g, carry2):
                r = g // 8
                coloff = (g % 8) * 16
                wfull = [wbuf[c, r, pl.ds(coloff, 16)] for c in range(8)]
                for lq in range(16):
                    q = g * 16 + lq
                    acc = jnp.zeros((16,), _F32)
                    for c in range(8):
                        acc = acc + _vperm(wfull[c], bcast[lq]) * rows[c, q, :]
                    accb[q, :] = acc
                return carry2

            lax.fori_loop(0, k // 16, group, 0)
            pltpu.sync_copy(accb, out_hbm.at[pl.ds(col, k), :])
            return carry

        lax.fori_loop(0, nchunk, chunk, 0)

    return body(grid8, qidx3, qw3)


def kernel(x_query_ref, x_source_ref, x_source_curr, W_trans, b_trans,
           W_dec1, b_dec1, W_dec2, b_dec2, W_out, b_out):
    nq = x_query_ref.shape[0]
    ns = x_source_ref.shape[0]
    npad = 1 << 14
    while npad < max(nq, ns):
        npad *= 2

    def cm(x, n):
        pad = jnp.full((npad - n, 3), 0.5, _F32)
        return jnp.concatenate([x, pad], 0).T.reshape(3, npad // 128, 128)

    xq3 = cm(x_query_ref, nq)
    xr3 = cm(x_source_ref, ns)
    xc3 = cm(x_source_curr, ns)

    w8 = jnp.zeros((8, 8), _F32).at[:3, :3].set(W_trans).at[3, :3].set(b_trans)
    qidx8, qw8, sb, rec = _prep(xq3, xr3, xc3, w8)

    sb_f = sb.reshape(npad)
    rec_f = rec.reshape(9, npad)

    grid16 = _sc_scatter(sb_f, rec_f, ns)

    raw_pm = _sc_sample(grid16, qidx8, qw8)
    raw_t = raw_pm.T[:8]

    w1a = jnp.zeros((64, 32), _F32).at[:, :24].set(W_dec1.T).at[:, 24].set(b_dec1)
    w2a = jnp.zeros((64, 72), _F32).at[:, :64].set(W_dec2.T).at[:, 64].set(b_dec2)
    woa = jnp.zeros((8, 72), _F32).at[:3, :64].set(W_out.T).at[:3, 64].set(b_out)
    qcoord = jnp.clip(x_query_ref, 0.0, 1.0).T
    qcoord = jnp.concatenate(
        [qcoord, jnp.full((3, npad - nq), 0.5, _F32)], axis=1)
    out8 = _decode(raw_t, qcoord, w1a, w2a, woa)
    return out8[:3, :nq].T


# speedup vs baseline: 5.6555x; 1.5594x over previous
import functools

import jax
import jax.numpy as jnp
from jax import lax
from jax.experimental import pallas as pl
from jax.experimental.pallas import tpu as pltpu
from jax.experimental.pallas import tpu_sc as plsc

GRID_RES = 128
NCELL = GRID_RES ** 3
_F32 = jnp.float32


def _corner_iw(base_flat, d, c):
    ox, oy, oz = (c >> 2) & 1, (c >> 1) & 1, c & 1
    idx = base_flat + (ox * GRID_RES * GRID_RES + oy * GRID_RES + oz)
    w = ((d[0] if ox else (1.0 - d[0]))
         * (d[1] if oy else (1.0 - d[1]))
         * (d[2] if oz else (1.0 - d[2])))
    return idx, w


def _prep_body(xq_ref, xr_ref, xc_ref, w_ref, qi_ref, qw_ref, sb_ref, rec_ref):
    r = GRID_RES - 1

    xq = jnp.clip(xq_ref[...], 0.0, 1.0)
    qc = xq * float(r)
    qb = jnp.minimum(jnp.floor(qc), float(r - 1))
    qd = qc - qb
    qbi = qb.astype(jnp.int32)
    qflat = (qbi[0] * (GRID_RES * GRID_RES) + qbi[1] * GRID_RES + qbi[2])
    qis, qws = [], []
    for c in range(8):
        i, w = _corner_iw(qflat, qd, c)
        qis.append(i)
        qws.append(w)
    qi_ref[...] = jnp.stack(qis, axis=0)
    qw_ref[...] = jnp.stack(qws, axis=0)

    xr = jnp.clip(xr_ref[...], 0.0, 1.0)
    u = xc_ref[...] - xr
    f = []
    for i in range(3):
        acc = w_ref[3, i]
        for j in range(3):
            acc = acc + u[j] * w_ref[j, i]
        f.append(jax.nn.gelu(acc))
    sc = jnp.clip(xr * float(r), 0.0, float(r) - 0.001)
    sbf = jnp.floor(sc)
    sd = sc - sbf
    sbi = sbf.astype(jnp.int32)
    sb_ref[...] = (sbi[0] * (GRID_RES * GRID_RES) + sbi[1] * GRID_RES + sbi[2])
    rec_ref[...] = jnp.stack(
        [f[0], f[1], f[2], u[0], u[1], u[2], sd[0], sd[1], sd[2]], axis=0)


def _prep(xq3, xr3, xc3, w8):
    nb = xq3.shape[1]
    bm = min(512, nb)
    grid = (nb // bm,)
    csp = pl.BlockSpec((3, bm, 128), lambda i: (0, i, 0))
    isp = pl.BlockSpec((bm, 128), lambda i: (i, 0))
    c8sp = pl.BlockSpec((8, bm, 128), lambda i: (0, i, 0))
    return pl.pallas_call(
        _prep_body,
        grid=grid,
        in_specs=[csp, csp, csp, pl.BlockSpec((8, 8), lambda i: (0, 0))],
        out_specs=[c8sp, c8sp, isp, pl.BlockSpec((9, bm, 128), lambda i: (0, i, 0))],
        out_shape=[
            jax.ShapeDtypeStruct((8, nb, 128), jnp.int32),
            jax.ShapeDtypeStruct((8, nb, 128), _F32),
            jax.ShapeDtypeStruct((nb, 128), jnp.int32),
            jax.ShapeDtypeStruct((9, nb, 128), _F32),
        ],
    )(xq3, xr3, xc3, w8)


def _decode_body(raw_ref, qf_ref, w1_ref, w2_ref, wo_ref, out_ref):
    raw = raw_ref[...]
    f = raw[0:3]
    dens = raw[3:4]
    u = raw[4:7]
    denom = jnp.maximum(dens, 1e-05)
    mask = (dens > 1e-05).astype(_F32)
    scale = mask / denom
    fn = f * scale
    un = u * scale
    qf = qf_ref[...]
    pe = []
    for i in range(3):
        freq = (2.0 ** i) * jnp.pi
        pe.append(jnp.sin(freq * qf))
        pe.append(jnp.cos(freq * qf))
    b = raw.shape[1]
    x = jnp.concatenate(
        [fn, un] + pe + [jnp.ones((1, b), _F32), jnp.zeros((7, b), _F32)],
        axis=0)
    h = jax.nn.gelu(jnp.dot(w1_ref[...], x, preferred_element_type=_F32))
    h = jnp.concatenate([h, jnp.ones((1, b), _F32), jnp.zeros((7, b), _F32)],
                        axis=0)
    h = jax.nn.gelu(jnp.dot(w2_ref[...], h, preferred_element_type=_F32))
    h = jnp.concatenate([h, jnp.ones((1, b), _F32), jnp.zeros((7, b), _F32)],
                        axis=0)
    o = jnp.dot(wo_ref[...], h, preferred_element_type=_F32)
    drift = o[0:3] + un
    out_ref[...] = jnp.clip(
        jnp.concatenate([drift, jnp.zeros((5, b), _F32)], axis=0), 0.001, 0.999)


def _decode(raw_t, qf3, w1a, w2a, woa):
    np_ = raw_t.shape[1]
    b = 2048
    grid = (np_ // b,)
    return pl.pallas_call(
        _decode_body,
        grid=grid,
        in_specs=[
            pl.BlockSpec((8, b), lambda i: (0, i)),
            pl.BlockSpec((3, b), lambda i: (0, i)),
            pl.BlockSpec((64, 32), lambda i: (0, 0)),
            pl.BlockSpec((64, 72), lambda i: (0, 0)),
            pl.BlockSpec((8, 72), lambda i: (0, 0)),
        ],
        out_specs=pl.BlockSpec((8, b), lambda i: (0, i)),
        out_shape=jax.ShapeDtypeStruct((8, np_), _F32),
    )(raw_t, qf3, w1a, w2a, woa)


def _vperm(v, idx):
    return lax.gather(
        v, idx[:, None],
        lax.GatherDimensionNumbers(offset_dims=(), collapsed_slice_dims=(0,),
                                   start_index_map=(0,)),
        (1,), mode=lax.GatherScatterMode.PROMISE_IN_BOUNDS)


_OFF = [0, 1, 128, 129, 16384, 16385, 16512, 16513]


def _sc_scatter(sb1d, rec2, n_src):
    npnt = sb1d.shape[0]
    per_t = npnt // 16
    k = min(2048, per_t)
    nchunk = per_t // k
    p_sz = 90880
    npart = 24
    npass = 12
    tail = NCELL - (npart - 1) * p_sz
    e = 1024
    zrows = (p_sz + 128) // 16
    mesh = plsc.VectorSubcoreMesh(core_axis_name="c", subcore_axis_name="s")

    @functools.partial(
        pl.kernel, mesh=mesh,
        out_type=jax.ShapeDtypeStruct((NCELL, 16), _F32),
        compiler_params=pltpu.CompilerParams(use_tc_tiling_on_sc=False,
                                             needs_layout_passes=False),
        scratch_types=[
            pltpu.VMEM((k,), jnp.int32),
            pltpu.VMEM((9, k), _F32),
            pltpu.VMEM((k + 16,), jnp.int32),
            pltpu.VMEM((e, 16), _F32),
            pltpu.VMEM((8, 128), jnp.int32),
            pltpu.VMEM_SHARED((p_sz + 128, 16), _F32),
            pltpu.SemaphoreType.DMA,
        ])
    def body(sb_hbm, rec_hbm, grid_hbm, idxs, chs, liveb, vals, eidx,
             accum, sem):
        ci = lax.axis_index("c")
        sid = lax.axis_index("s")
        lane = lax.iota(jnp.int32, 16)
        dump = [jnp.full((16,), p_sz + c * 16, jnp.int32) + lane
                for c in range(8)]

        def one_pass(pa, carry0):
            p = pa * 2 + ci
            lo = p * p_sz
            hi = jnp.minimum(lo + p_sz, NCELL)
            psize = hi - lo

            @pl.when(p < npart)
            def _run():
                def vrow(i, carry):
                    vals[i, :] = jnp.zeros((16,), _F32)
                    return carry
                lax.fori_loop(0, e, vrow, 0)
                zs = [pltpu.async_copy(
                    vals.at[pl.ds(0, 1024), :],
                    accum.at[pl.ds(sid * zrows + i * 1024, 1024), :], sem)
                    for i in range(5)]
                zs.append(pltpu.async_copy(
                    vals.at[pl.ds(0, 568), :],
                    accum.at[pl.ds(sid * zrows + 5120, 568), :], sem))
                for d_ in zs:
                    d_.wait()
                plsc.subcore_barrier()

                def chunk(ci2, carry):
                    cstart = sid * per_t + ci2 * k
                    stage = [pltpu.async_copy(sb_hbm.at[pl.ds(cstart, k)],
                                              idxs, sem)]
                    stage += [
                        pltpu.async_copy(rec_hbm.at[c].at[pl.ds(cstart, k)],
                                         chs.at[c], sem)
                        for c in range(9)]
                    for d_ in stage:
                        d_.wait()

                    def scan(g, nlive):
                        bv = idxs[pl.ds(g * 16, 16)]
                        ordv = cstart + g * 16 + lane
                        live = ((bv >= lo - 16513) & (bv < hi)
                                & (ordv < n_src))
                        cs = plsc.cumsum(jnp.where(live, 1, 0))
                        plsc.store_scatter(liveb, [nlive + cs - 1],
                                           g * 16 + lane, mask=live)
                        return nlive + jnp.sum(jnp.where(live, 1, 0))

                    nlive = lax.fori_loop(0, k // 16, scan, 0)
                    ntrip = (nlive + 15) // 16

                    def livegrp(t, carry2):
                        ordv = liveb[pl.ds(t * 16, 16)]
                        lmask = (t * 16 + lane) < nlive
                        ordv = jnp.where(lmask, ordv, 0)
                        bvl = plsc.load_gather(idxs, [ordv]) - lo
                        ch = [plsc.load_gather(chs.at[c], [ordv])
                              for c in range(9)]
                        dx, dy, dz = ch[6], ch[7], ch[8]
                        wx0, wy0, wz0 = 1.0 - dx, 1.0 - dy, 1.0 - dz
                        wxy = [wx0 * wy0, wx0 * dy, dx * wy0, dx * dy]
                        blk = t & 7
                        blkv = jnp.full((16,), blk, jnp.int32)
                        for c in range(8):
                            w = wxy[c >> 1] * (dz if (c & 1) else wz0)
                            cell = bvl + _OFF[c]
                            inm = (cell >= 0) & (cell < psize) & lmask
                            cellw = jnp.where(inm, cell, dump[c])
                            col = jnp.full((16,), c * 16, jnp.int32) + lane
                            plsc.store_scatter(eidx, [blkv, col], cellw)
                            pos = blkv * 128 + col
                            vj = [ch[0] * w, ch[1] * w, ch[2] * w, w,
                                  ch[3] * w, ch[4] * w, ch[5] * w]
                            for j in range(7):
                                plsc.store_scatter(
                                    vals, [pos, jnp.full((16,), j, jnp.int32)],
                                    vj[j])

                        @pl.when(blk == 7)
                        def _flush():
                            ds_ = [pltpu.async_copy(
                                vals.at[pl.ds(f * 128, 128), :],
                                accum.at[eidx.at[f]], sem, add=True)
                                for f in range(8)]
                            for d_ in ds_:
                                d_.wait()
                        return carry2

                    lax.fori_loop(0, ntrip, livegrp, 0)

                    def fflush(f, carry3):
                        pltpu.sync_copy(vals.at[pl.ds(f * 128, 128), :],
                                        accum.at[eidx.at[f]], add=True)
                        return carry3

                    lax.fori_loop(0, ntrip & 7, fflush, 0)
                    return carry

                lax.fori_loop(0, nchunk, chunk, 0)
                plsc.subcore_barrier()

                @pl.when(p == npart - 1)
                def _dlast():
                    tr = tail // 16
                    pltpu.sync_copy(
                        accum.at[pl.ds(sid * tr, tr), :],
                        grid_hbm.at[pl.ds(lo + sid * tr, tr), :])

                @pl.when(p < npart - 1)
                def _dfull():
                    ds_ = [pltpu.async_copy(
                        accum.at[pl.ds(sid * 5680 + i * 1024, 1024), :],
                        grid_hbm.at[pl.ds(lo + sid * 5680 + i * 1024,
                                          1024), :], sem)
                        for i in range(5)]
                    ds_.append(pltpu.async_copy(
                        accum.at[pl.ds(sid * 5680 + 5120, 560), :],
                        grid_hbm.at[pl.ds(lo + sid * 5680 + 5120, 560), :],
                        sem))
                    for d_ in ds_:
                        d_.wait()
            return carry0

        lax.fori_loop(0, npass, one_pass, 0)

    return body(sb1d, rec2)
def _sc_sample(grid8, qidx3, qw3):
    nb = qidx3.shape[1]
    npnt = nb * 128
    nc, ns = 2, 16
    nw = nc * ns
    per_w = npnt // nw
    k = min(512, per_w)
    kr = k // 128
    nchunk = per_w // k
    mesh = plsc.VectorSubcoreMesh(core_axis_name="c", subcore_axis_name="s")

    @functools.partial(
        pl.kernel, mesh=mesh,
        out_type=jax.ShapeDtypeStruct((npnt, 16), _F32),
        compiler_params=pltpu.CompilerParams(use_tc_tiling_on_sc=False),
        scratch_types=[
            pltpu.VMEM((8, kr, 128), jnp.int32),
            pltpu.VMEM((8, kr, 128), _F32),
            pltpu.VMEM((8, k, 16), _F32),
            pltpu.VMEM((k, 16), _F32),
            pltpu.SemaphoreType.DMA,
        ])
    def body(grid_hbm, qi_hbm, qw_hbm, out_hbm, idxb, wbuf, rows, accb, sem):
        wid = lax.axis_index("s") * nc + lax.axis_index("c")
        base = wid * per_w
        bcast = [jnp.full((16,), l, jnp.int32) for l in range(16)]

        def chunk(ci, carry):
            col = base + ci * k
            rb = pl.multiple_of(col // 128, kr)
            stage = []
            for c in range(8):
                stage.append(pltpu.async_copy(qi_hbm.at[c, pl.ds(rb, kr), :],
                                              idxb.at[c], sem))
                stage.append(pltpu.async_copy(qw_hbm.at[c, pl.ds(rb, kr), :],
                                              wbuf.at[c], sem))
            for d_ in stage:
                d_.wait()
            descs = [pltpu.async_copy(grid_hbm.at[idxb.at[c, s]],
                                      rows.at[c, pl.ds(s * 128, 128), :], sem)
                     for c in range(8) for s in range(kr)]
            for d_ in descs:
                d_.wait()

            def group(g, carry2):
                r = g // 8
                coloff = (g % 8) * 16
                wfull = [wbuf[c, r, pl.ds(coloff, 16)] for c in range(8)]
                for lq in range(16):
                    q = g * 16 + lq
                    acc = jnp.zeros((16,), _F32)
                    for c in range(8):
                        acc = acc + _vperm(wfull[c], bcast[lq]) * rows[c, q, :]
                    accb[q, :] = acc
                return carry2

            lax.fori_loop(0, k // 16, group, 0)
            pltpu.sync_copy(accb, out_hbm.at[pl.ds(col, k), :])
            return carry

        lax.fori_loop(0, nchunk, chunk, 0)

    return body(grid8, qidx3, qw3)


def kernel(x_query_ref, x_source_ref, x_source_curr, W_trans, b_trans,
           W_dec1, b_dec1, W_dec2, b_dec2, W_out, b_out):
    nq = x_query_ref.shape[0]
    ns = x_source_ref.shape[0]
    npad = 1 << 14
    while npad < max(nq, ns):
        npad *= 2

    def cm(x, n):
        pad = jnp.full((npad - n, 3), 0.5, _F32)
        return jnp.concatenate([x, pad], 0).T.reshape(3, npad // 128, 128)

    xq3 = cm(x_query_ref, nq)
    xr3 = cm(x_source_ref, ns)
    xc3 = cm(x_source_curr, ns)

    w8 = jnp.zeros((8, 8), _F32).at[:3, :3].set(W_trans).at[3, :3].set(b_trans)
    qidx8, qw8, sb, rec = _prep(xq3, xr3, xc3, w8)

    sb_f = sb.reshape(npad)
    rec_f = rec.reshape(9, npad)

    grid16 = _sc_scatter(sb_f, rec_f, ns)

    raw_pm = _sc_sample(grid16, qidx8, qw8)
    raw_t = raw_pm.T[:8]

    w1a = jnp.zeros((64, 32), _F32).at[:, :24].set(W_dec1.T).at[:, 24].set(b_dec1)
    w2a = jnp.zeros((64, 72), _F32).at[:, :64].set(W_dec2.T).at[:, 64].set(b_dec2)
    woa = jnp.zeros((8, 72), _F32).at[:3, :64].set(W_out.T).at[:3, 64].set(b_out)
    qcoord = jnp.clip(x_query_ref, 0.0, 1.0).T
    qcoord = jnp.concatenate(
        [qcoord, jnp.full((3, npad - nq), 0.5, _F32)], axis=1)
    out8 = _decode(raw_t, qcoord, w1a, w2a, woa)
    return out8[:3, :nq].T
